# baseline (speedup 1.0000x reference)
import functools

import jax
import jax.numpy as jnp
from jax import lax
from jax.experimental import pallas as pl
from jax.experimental.pallas import tpu as pltpu
from jax.experimental.pallas import tpu_sc as plsc

_BATCH = 16384
_EMB = 64

_NC = 2
_NS = 16
_NW = _NC * _NS
_BPW = _BATCH // _NW
_CH = 128
_NCHUNK = _BPW // _CH


def _gather_body(u_hbm, v_hbm, user_hbm, item_hbm, urows_out, vrows_out,
                 uidx, vidx, urows, vrows, sem):
    wid = lax.axis_index("s") * _NC + lax.axis_index("c")
    base = wid * _BPW
    pltpu.sync_copy(u_hbm.at[wid], uidx)
    pltpu.sync_copy(v_hbm.at[wid], vidx)
    copies = []
    for j in range(_NCHUNK):
        copies.append(pltpu.async_copy(
            user_hbm.at[uidx.at[j]], urows.at[pl.ds(j * _CH, _CH)], sem))
    for j in range(_NCHUNK):
        copies.append(pltpu.async_copy(
            item_hbm.at[vidx.at[j]], vrows.at[pl.ds(j * _CH, _CH)], sem))
    for c in copies:
        c.wait()
    pltpu.sync_copy(urows, urows_out.at[pl.ds(base, _BPW)])
    pltpu.sync_copy(vrows, vrows_out.at[pl.ds(base, _BPW)])


_sc_gather = functools.partial(
    pl.kernel,
    out_type=[jax.ShapeDtypeStruct((_BATCH, _EMB), jnp.float32),
              jax.ShapeDtypeStruct((_BATCH, _EMB), jnp.float32)],
    mesh=plsc.VectorSubcoreMesh(core_axis_name="c", subcore_axis_name="s"),
    compiler_params=pltpu.CompilerParams(use_tc_tiling_on_sc=False),
    scratch_types=[
        pltpu.VMEM((_NCHUNK, _CH), jnp.int32),
        pltpu.VMEM((_NCHUNK, _CH), jnp.int32),
        pltpu.VMEM((_BPW, _EMB), jnp.float32),
        pltpu.VMEM((_BPW, _EMB), jnp.float32),
        pltpu.SemaphoreType.DMA,
    ],
)(_gather_body)


_BM = 2048


def _mlp_body(u_ref, v_ref, w1u_ref, w1v_ref, b1_ref, w2_ref, b2_ref, o_ref):
    zu = jnp.maximum(u_ref[...], 0.0)
    zv = jnp.maximum(v_ref[...], 0.0)
    h = (jnp.dot(zu, w1u_ref[...], preferred_element_type=jnp.float32)
         + jnp.dot(zv, w1v_ref[...], preferred_element_type=jnp.float32)
         + b1_ref[...])
    h = jnp.maximum(h, 0.0)
    o_ref[...] = (jnp.dot(h, w2_ref[...], preferred_element_type=jnp.float32)
                  + b2_ref[...])


def _mlp(urows, vrows, w1u, w1v, b1, w2, b2):
    return pl.pallas_call(
        _mlp_body,
        grid=(_BATCH // _BM,),
        in_specs=[
            pl.BlockSpec((_BM, _EMB), lambda i: (i, 0)),
            pl.BlockSpec((_BM, _EMB), lambda i: (i, 0)),
            pl.BlockSpec((_EMB, _EMB), lambda i: (0, 0)),
            pl.BlockSpec((_EMB, _EMB), lambda i: (0, 0)),
            pl.BlockSpec((1, _EMB), lambda i: (0, 0)),
            pl.BlockSpec((_EMB, 1), lambda i: (0, 0)),
            pl.BlockSpec((1, 1), lambda i: (0, 0)),
        ],
        out_specs=pl.BlockSpec((_BM, 1), lambda i: (i, 0)),
        out_shape=jax.ShapeDtypeStruct((_BATCH, 1), jnp.float32),
    )(urows, vrows, w1u, w1v, b1, w2, b2)


def kernel(u, v, user_emb, item_emb, W1, b1, W2, b2):
    u3 = u.astype(jnp.int32).reshape(_NW, _NCHUNK, _CH)
    v3 = v.astype(jnp.int32).reshape(_NW, _NCHUNK, _CH)
    urows, vrows = _sc_gather(u3, v3, user_emb, item_emb)
    return _mlp(urows, vrows, W1[:_EMB], W1[_EMB:], b1.reshape(1, _EMB),
                W2, b2.reshape(1, 1))
